# Initial kernel scaffold; baseline (speedup 1.0000x reference)
import jax
import jax.numpy as jnp
from jax.experimental import pallas as pl


def kernel(encoded_trajectories, W, b, gamma, beta, moving_mean, moving_var):
    raise NotImplementedError("write your pallas kernel here")


# speedup vs baseline: 1.3355x; 1.3355x over previous
import jax
import jax.numpy as jnp
from jax.experimental import pallas as pl

_BLOCK_ROWS = 2048


def _body(x_ref, w_ref, b_ref, gamma_ref, beta_ref, mean_ref, var_ref, o_ref):
    eps = 1e-3
    scale = gamma_ref[:] * jax.lax.rsqrt(var_ref[:] + eps)
    shift = (b_ref[:] - mean_ref[:]) * scale + beta_ref[:]
    acc = jnp.dot(x_ref[:], w_ref[:], preferred_element_type=jnp.float32)
    o_ref[:] = acc * scale + shift


def kernel(encoded_trajectories, W, b, gamma, beta, moving_mean, moving_var):
    x = encoded_trajectories
    B, N, D = x.shape
    U = W.shape[1]
    rows = B * N
    x2 = x.reshape(rows, D)

    vec = lambda v: v.reshape(1, U)
    br = _BLOCK_ROWS
    grid = (rows // br,)

    out = pl.pallas_call(
        _body,
        grid=grid,
        in_specs=[
            pl.BlockSpec((br, D), lambda i: (i, 0)),
            pl.BlockSpec((D, U), lambda i: (0, 0)),
            pl.BlockSpec((1, U), lambda i: (0, 0)),
            pl.BlockSpec((1, U), lambda i: (0, 0)),
            pl.BlockSpec((1, U), lambda i: (0, 0)),
            pl.BlockSpec((1, U), lambda i: (0, 0)),
            pl.BlockSpec((1, U), lambda i: (0, 0)),
        ],
        out_specs=pl.BlockSpec((br, U), lambda i: (i, 0)),
        out_shape=jax.ShapeDtypeStruct((rows, U), x.dtype),
    )(x2, W, vec(b), vec(gamma), vec(beta), vec(moving_mean), vec(moving_var))

    return out.reshape(B, N, U)


# speedup vs baseline: 2.0890x; 1.5642x over previous
import jax
import jax.numpy as jnp
from jax.experimental import pallas as pl

_BLOCK_ROWS = 8192


def _body(x_ref, w_ref, b_ref, gamma_ref, beta_ref, mean_ref, var_ref, o_ref):
    eps = 1e-3
    scale = gamma_ref[:] * jax.lax.rsqrt(var_ref[:] + eps)
    shift = (b_ref[:] - mean_ref[:]) * scale + beta_ref[:]
    acc = jnp.dot(x_ref[:], w_ref[:], preferred_element_type=jnp.float32)
    o_ref[:] = acc * scale + shift


def kernel(encoded_trajectories, W, b, gamma, beta, moving_mean, moving_var):
    x = encoded_trajectories
    B, N, D = x.shape
    U = W.shape[1]
    rows = B * N
    x2 = x.reshape(rows, D)

    vec = lambda v: v.reshape(1, U)
    br = _BLOCK_ROWS
    grid = (rows // br,)

    out = pl.pallas_call(
        _body,
        grid=grid,
        in_specs=[
            pl.BlockSpec((br, D), lambda i: (i, 0)),
            pl.BlockSpec((D, U), lambda i: (0, 0)),
            pl.BlockSpec((1, U), lambda i: (0, 0)),
            pl.BlockSpec((1, U), lambda i: (0, 0)),
            pl.BlockSpec((1, U), lambda i: (0, 0)),
            pl.BlockSpec((1, U), lambda i: (0, 0)),
            pl.BlockSpec((1, U), lambda i: (0, 0)),
        ],
        out_specs=pl.BlockSpec((br, U), lambda i: (i, 0)),
        out_shape=jax.ShapeDtypeStruct((rows, U), x.dtype),
    )(x2, W, vec(b), vec(gamma), vec(beta), vec(moving_mean), vec(moving_var))

    return out.reshape(B, N, U)


# speedup vs baseline: 2.2225x; 1.0639x over previous
import jax
import jax.numpy as jnp
from jax.experimental import pallas as pl

_BLOCK_ROWS = 16384


def _body(x_ref, w_ref, b_ref, gamma_ref, beta_ref, mean_ref, var_ref, o_ref):
    eps = 1e-3
    scale = gamma_ref[:] * jax.lax.rsqrt(var_ref[:] + eps)
    shift = (b_ref[:] - mean_ref[:]) * scale + beta_ref[:]
    acc = jnp.dot(x_ref[:], w_ref[:], preferred_element_type=jnp.float32)
    o_ref[:] = acc * scale + shift


def kernel(encoded_trajectories, W, b, gamma, beta, moving_mean, moving_var):
    x = encoded_trajectories
    B, N, D = x.shape
    U = W.shape[1]
    rows = B * N
    x2 = x.reshape(rows, D)

    vec = lambda v: v.reshape(1, U)
    br = _BLOCK_ROWS
    grid = (rows // br,)

    out = pl.pallas_call(
        _body,
        grid=grid,
        in_specs=[
            pl.BlockSpec((br, D), lambda i: (i, 0)),
            pl.BlockSpec((D, U), lambda i: (0, 0)),
            pl.BlockSpec((1, U), lambda i: (0, 0)),
            pl.BlockSpec((1, U), lambda i: (0, 0)),
            pl.BlockSpec((1, U), lambda i: (0, 0)),
            pl.BlockSpec((1, U), lambda i: (0, 0)),
            pl.BlockSpec((1, U), lambda i: (0, 0)),
        ],
        out_specs=pl.BlockSpec((br, U), lambda i: (i, 0)),
        out_shape=jax.ShapeDtypeStruct((rows, U), x.dtype),
    )(x2, W, vec(b), vec(gamma), vec(beta), vec(moving_mean), vec(moving_var))

    return out.reshape(B, N, U)
